# speedup vs baseline: 2.0528x; 2.0528x over previous
import jax
import jax.numpy as jnp
from jax import lax
from jax.experimental import pallas as pl
from jax.experimental.pallas import tpu as pltpu
from jax.experimental.pallas import tpu_sc as plsc

VOCAB = 100000
MAX_LEN = 200
DIM = 128
BATCH = 1024

NC = 2
NS = 16
NW = NC * NS
ROWS_PER_W = BATCH // NW
IDX_SPLIT = 2
IDX_CHUNK = MAX_LEN // IDX_SPLIT


def _build_pe():
    pos = jnp.arange(MAX_LEN, dtype=jnp.float32).reshape(-1, 1)
    div = jnp.power(
        10000.0, jnp.arange(0, DIM, 2, dtype=jnp.float32) / DIM)
    ang = pos / div
    pe = jnp.zeros((MAX_LEN, DIM), dtype=jnp.float32)
    pe = pe.at[:, 0::2].set(jnp.sin(ang))
    pe = pe.at[:, 1::2].set(jnp.cos(ang))
    return pe


def _sc_body(x_hbm, pe_hbm, table_hbm, out_hbm, idx_v, rows_v, pe_v, sem):
    wid = lax.axis_index("s") * NC + lax.axis_index("c")
    pltpu.sync_copy(pe_hbm, pe_v)

    def row_body(t, carry):
        r = wid * ROWS_PER_W + t
        pltpu.sync_copy(x_hbm.at[r], idx_v)
        cps = []
        for j in range(IDX_SPLIT):
            cps.append(pltpu.async_copy(
                table_hbm.at[idx_v.at[j]],
                rows_v.at[pl.ds(j * IDX_CHUNK, IDX_CHUNK)],
                sem,
            ))
        for cp in cps:
            cp.wait()

        def add_body(i, c):
            for cc in range(DIM // 16):
                sl = pl.ds(cc * 16, 16)
                rows_v[i, sl] = rows_v[i, sl] + pe_v[i, sl]
            return c

        lax.fori_loop(0, MAX_LEN, add_body, 0, unroll=2)
        pltpu.sync_copy(rows_v, out_hbm.at[pl.ds(r * MAX_LEN, MAX_LEN)])
        return carry

    lax.fori_loop(0, ROWS_PER_W, row_body, 0)


@jax.jit
def _run(x3, pe, table):
    mesh = plsc.VectorSubcoreMesh(
        core_axis_name="c", subcore_axis_name="s",
        num_cores=NC, num_subcores=NS)
    f = pl.kernel(
        _sc_body,
        out_type=jax.ShapeDtypeStruct((BATCH * MAX_LEN, DIM), jnp.float32),
        mesh=mesh,
        scratch_types=[
            pltpu.VMEM((IDX_SPLIT, IDX_CHUNK), jnp.int32),
            pltpu.VMEM((MAX_LEN, DIM), jnp.float32),
            pltpu.VMEM((MAX_LEN, DIM), jnp.float32),
            pltpu.SemaphoreType.DMA,
        ],
    )
    return f(x3, pe, table)


def kernel(x, table):
    x3 = x.reshape(BATCH, IDX_SPLIT, IDX_CHUNK)
    pe = _build_pe()
    out = _run(x3, pe, table)
    return out.reshape(BATCH, MAX_LEN, DIM)


# speedup vs baseline: 6.2166x; 3.0283x over previous
import jax
import jax.numpy as jnp
from jax import lax
from jax.experimental import pallas as pl
from jax.experimental.pallas import tpu as pltpu
from jax.experimental.pallas import tpu_sc as plsc

VOCAB = 100000
MAX_LEN = 200
DIM = 128
BATCH = 1024

NC = 2
NS = 16
NW = NC * NS
ROWS_PER_W = BATCH // NW
IDX_SPLIT = 2
IDX_CHUNK = MAX_LEN // IDX_SPLIT


def _build_pe():
    pos = jnp.arange(MAX_LEN, dtype=jnp.float32).reshape(-1, 1)
    div = jnp.power(
        10000.0, jnp.arange(0, DIM, 2, dtype=jnp.float32) / DIM)
    ang = pos / div
    pe = jnp.zeros((MAX_LEN, DIM), dtype=jnp.float32)
    pe = pe.at[:, 0::2].set(jnp.sin(ang))
    pe = pe.at[:, 1::2].set(jnp.cos(ang))
    return pe


def _sc_body(x_hbm, pe_hbm, table_hbm, out_hbm,
             idx_all, rows0, rows1, pe_v, gsem0, gsem1, ssem0, ssem1):
    wid = lax.axis_index("s") * NC + lax.axis_index("c")
    base_row = wid * ROWS_PER_W
    rows = (rows0, rows1)
    gsem = (gsem0, gsem1)
    ssem = (ssem0, ssem1)

    pltpu.sync_copy(pe_hbm, pe_v)
    pltpu.sync_copy(x_hbm.at[pl.ds(base_row, ROWS_PER_W)], idx_all)

    pend_gather = [None, None]
    pend_scatter = [None, None]

    def fire_gather(t, b):
        pend_gather[b] = [
            pltpu.async_copy(
                table_hbm.at[idx_all.at[t, j]],
                rows[b].at[pl.ds(j * IDX_CHUNK, IDX_CHUNK)],
                gsem[b],
            )
            for j in range(IDX_SPLIT)
        ]

    fire_gather(0, 0)
    for t in range(ROWS_PER_W):
        b = t & 1
        nb = 1 - b
        if t + 1 < ROWS_PER_W:
            if pend_scatter[nb] is not None:
                pend_scatter[nb].wait()
                pend_scatter[nb] = None
            fire_gather(t + 1, nb)
        for cp in pend_gather[b]:
            cp.wait()

        rows_b = rows[b]

        def add_body(i, c):
            for cc in range(DIM // 16):
                sl = pl.ds(cc * 16, 16)
                plsc.addupdate(rows_b.at[i, sl], pe_v[i, sl])
            return c

        lax.fori_loop(0, MAX_LEN, add_body, 0, unroll=2)
        pend_scatter[b] = pltpu.async_copy(
            rows_b, out_hbm.at[pl.ds((base_row + t) * MAX_LEN, MAX_LEN)],
            ssem[b])
    for b in range(2):
        if pend_scatter[b] is not None:
            pend_scatter[b].wait()


@jax.jit
def _run(x3, pe, table):
    mesh = plsc.VectorSubcoreMesh(
        core_axis_name="c", subcore_axis_name="s",
        num_cores=NC, num_subcores=NS)
    f = pl.kernel(
        _sc_body,
        out_type=jax.ShapeDtypeStruct((BATCH * MAX_LEN, DIM), jnp.float32),
        mesh=mesh,
        scratch_types=[
            pltpu.VMEM((ROWS_PER_W, IDX_SPLIT, IDX_CHUNK), jnp.int32),
            pltpu.VMEM((MAX_LEN, DIM), jnp.float32),
            pltpu.VMEM((MAX_LEN, DIM), jnp.float32),
            pltpu.VMEM((MAX_LEN, DIM), jnp.float32),
            pltpu.SemaphoreType.DMA,
            pltpu.SemaphoreType.DMA,
            pltpu.SemaphoreType.DMA,
            pltpu.SemaphoreType.DMA,
        ],
    )
    return f(x3, pe, table)


def kernel(x, table):
    x3 = x.reshape(BATCH, IDX_SPLIT, IDX_CHUNK)
    pe = _build_pe()
    out = _run(x3, pe, table)
    return out.reshape(BATCH, MAX_LEN, DIM)


# speedup vs baseline: 6.8830x; 1.1072x over previous
import jax
import jax.numpy as jnp
from jax import lax
from jax.experimental import pallas as pl
from jax.experimental.pallas import tpu as pltpu
from jax.experimental.pallas import tpu_sc as plsc

VOCAB = 100000
MAX_LEN = 200
DIM = 128
BATCH = 1024

NC = 2
NS = 16
NW = NC * NS
STAGE = 128
STAGES = (BATCH // NW) * MAX_LEN // STAGE
NBUF = 4
LOOKAHEAD = 2


def _build_pe():
    pos = jnp.arange(MAX_LEN, dtype=jnp.float32).reshape(-1, 1)
    div = jnp.power(
        10000.0, jnp.arange(0, DIM, 2, dtype=jnp.float32) / DIM)
    ang = pos / div
    pe = jnp.zeros((MAX_LEN, DIM), dtype=jnp.float32)
    pe = pe.at[:, 0::2].set(jnp.sin(ang))
    pe = pe.at[:, 1::2].set(jnp.cos(ang))
    return pe


def _sc_body(x_hbm, pe_hbm, table_hbm, out_hbm,
             idx_all, r0, r1, r2, r3, pe_v,
             g0, g1, g2, g3, s0, s1, s2, s3):
    wid = lax.axis_index("s") * NC + lax.axis_index("c")
    rows = (r0, r1, r2, r3)
    gsem = (g0, g1, g2, g3)
    ssem = (s0, s1, s2, s3)
    out_base = wid * (STAGES * STAGE)

    pltpu.sync_copy(pe_hbm, pe_v)
    pltpu.sync_copy(x_hbm.at[wid], idx_all)

    pend_g = [None] * NBUF
    pend_s = [None] * NBUF

    def fire_gather(s):
        b = s % NBUF
        pend_g[b] = pltpu.async_copy(
            table_hbm.at[idx_all.at[s]], rows[b], gsem[b])

    for s in range(LOOKAHEAD):
        fire_gather(s)
    for s in range(STAGES):
        b = s % NBUF
        if s + LOOKAHEAD < STAGES:
            b2 = (s + LOOKAHEAD) % NBUF
            if pend_s[b2] is not None:
                pend_s[b2].wait()
                pend_s[b2] = None
            fire_gather(s + LOOKAHEAD)
        pend_g[b].wait()

        rows_b = rows[b]
        base = (s * STAGE) % MAX_LEN
        cut = min(STAGE, MAX_LEN - base)

        def run_add(lo, hi, off):
            def add_body(i, c):
                for cc in range(DIM // 16):
                    sl = pl.ds(cc * 16, 16)
                    plsc.addupdate(rows_b.at[i, sl], pe_v[i + off, sl])
                return c
            lax.fori_loop(lo, hi, add_body, 0, unroll=2)

        run_add(0, cut, base)
        if cut < STAGE:
            run_add(cut, STAGE, base - MAX_LEN)
        pend_s[b] = pltpu.async_copy(
            rows_b, out_hbm.at[pl.ds(out_base + s * STAGE, STAGE)], ssem[b])
    for b in range(NBUF):
        if pend_s[b] is not None:
            pend_s[b].wait()


@jax.jit
def _run(x2, pe, table):
    mesh = plsc.VectorSubcoreMesh(
        core_axis_name="c", subcore_axis_name="s",
        num_cores=NC, num_subcores=NS)
    f = pl.kernel(
        _sc_body,
        out_type=jax.ShapeDtypeStruct((BATCH * MAX_LEN, DIM), jnp.float32),
        mesh=mesh,
        scratch_types=[
            pltpu.VMEM((STAGES, STAGE), jnp.int32),
            pltpu.VMEM((STAGE, DIM), jnp.float32),
            pltpu.VMEM((STAGE, DIM), jnp.float32),
            pltpu.VMEM((STAGE, DIM), jnp.float32),
            pltpu.VMEM((STAGE, DIM), jnp.float32),
            pltpu.VMEM((MAX_LEN, DIM), jnp.float32),
            pltpu.SemaphoreType.DMA,
            pltpu.SemaphoreType.DMA,
            pltpu.SemaphoreType.DMA,
            pltpu.SemaphoreType.DMA,
            pltpu.SemaphoreType.DMA,
            pltpu.SemaphoreType.DMA,
            pltpu.SemaphoreType.DMA,
            pltpu.SemaphoreType.DMA,
        ],
    )
    return f(x2, pe, table)


def kernel(x, table):
    x2 = x.reshape(NW, STAGES, STAGE)
    pe = _build_pe()
    out = _run(x2, pe, table)
    return out.reshape(BATCH, MAX_LEN, DIM)


# speedup vs baseline: 6.9166x; 1.0049x over previous
import jax
import jax.numpy as jnp
from jax import lax
from jax.experimental import pallas as pl
from jax.experimental.pallas import tpu as pltpu
from jax.experimental.pallas import tpu_sc as plsc

VOCAB = 100000
MAX_LEN = 200
DIM = 128
BATCH = 1024

NC = 2
NS = 16
NW = NC * NS
STAGE = 128
STAGES = (BATCH // NW) * MAX_LEN // STAGE
NBUF = 5
LOOKAHEAD = 3
HALF = STAGE // 2


def _build_pe():
    pos = jnp.arange(MAX_LEN, dtype=jnp.float32).reshape(-1, 1)
    div = jnp.power(
        10000.0, jnp.arange(0, DIM, 2, dtype=jnp.float32) / DIM)
    ang = pos / div
    pe = jnp.zeros((MAX_LEN, DIM), dtype=jnp.float32)
    pe = pe.at[:, 0::2].set(jnp.sin(ang))
    pe = pe.at[:, 1::2].set(jnp.cos(ang))
    return pe


def _sc_body(x_hbm, pe_hbm, table_hbm, out_hbm,
             idx_all, r0, r1, r2, r3, r4, pe_v,
             g0, g1, g2, g3, g4, s0, s1, s2, s3, s4):
    wid = lax.axis_index("s") * NC + lax.axis_index("c")
    rows = (r0, r1, r2, r3, r4)
    gsem = (g0, g1, g2, g3, g4)
    ssem = (s0, s1, s2, s3, s4)
    out_base = wid * (STAGES * STAGE)

    pltpu.sync_copy(pe_hbm, pe_v)
    pltpu.sync_copy(x_hbm.at[wid], idx_all)

    BLOCK = 25

    def block_body(k, carry):
        s0_dyn = k * (BLOCK * STAGE)
        pend_g = [None] * NBUF
        pend_s = [[] for _ in range(NBUF)]

        def fire_gather(j):
            b = j % NBUF
            pend_g[b] = pltpu.async_copy(
                table_hbm.at[idx_all.at[k * BLOCK + j]], rows[b], gsem[b])

        for j in range(LOOKAHEAD):
            fire_gather(j)
        for j in range(BLOCK):
            b = j % NBUF
            if j + LOOKAHEAD < BLOCK:
                b2 = (j + LOOKAHEAD) % NBUF
                for cp in pend_s[b2]:
                    cp.wait()
                pend_s[b2] = []
                fire_gather(j + LOOKAHEAD)
            pend_g[b].wait()

            rows_b = rows[b]
            base = (j * STAGE) % MAX_LEN
            cut = min(STAGE, MAX_LEN - base)

            def run_add(lo, hi):
                segs = []
                if lo < cut:
                    segs.append((lo, min(hi, cut), base))
                if hi > cut:
                    segs.append((max(lo, cut), hi, base - MAX_LEN))
                for (l, h, off) in segs:
                    @plsc.parallel_loop(l, h, unroll=2)
                    def add_body(i, off=off, rows_b=rows_b):
                        for cc in range(DIM // 16):
                            sl = pl.ds(cc * 16, 16)
                            plsc.addupdate(
                                rows_b.at[i, sl], pe_v[i + off, sl])

            for h in range(2):
                run_add(h * HALF, (h + 1) * HALF)
                pend_s[b].append(pltpu.async_copy(
                    rows_b.at[pl.ds(h * HALF, HALF)],
                    out_hbm.at[pl.ds(
                        out_base + s0_dyn + j * STAGE + h * HALF, HALF)],
                    ssem[b]))
        for b in range(NBUF):
            for cp in pend_s[b]:
                cp.wait()
        return carry

    lax.fori_loop(0, STAGES // BLOCK, block_body, 0)


@jax.jit
def _run(x2, pe, table):
    mesh = plsc.VectorSubcoreMesh(
        core_axis_name="c", subcore_axis_name="s",
        num_cores=NC, num_subcores=NS)
    f = pl.kernel(
        _sc_body,
        out_type=jax.ShapeDtypeStruct((BATCH * MAX_LEN, DIM), jnp.float32),
        mesh=mesh,
        scratch_types=[
            pltpu.VMEM((STAGES, STAGE), jnp.int32),
            pltpu.VMEM((STAGE, DIM), jnp.float32),
            pltpu.VMEM((STAGE, DIM), jnp.float32),
            pltpu.VMEM((STAGE, DIM), jnp.float32),
            pltpu.VMEM((STAGE, DIM), jnp.float32),
            pltpu.VMEM((STAGE, DIM), jnp.float32),
            pltpu.VMEM((MAX_LEN, DIM), jnp.float32),
            pltpu.SemaphoreType.DMA,
            pltpu.SemaphoreType.DMA,
            pltpu.SemaphoreType.DMA,
            pltpu.SemaphoreType.DMA,
            pltpu.SemaphoreType.DMA,
            pltpu.SemaphoreType.DMA,
            pltpu.SemaphoreType.DMA,
            pltpu.SemaphoreType.DMA,
            pltpu.SemaphoreType.DMA,
            pltpu.SemaphoreType.DMA,
        ],
    )
    return f(x2, pe, table)


def kernel(x, table):
    x2 = x.reshape(NW, STAGES, STAGE)
    pe = _build_pe()
    out = _run(x2, pe, table)
    return out.reshape(BATCH, MAX_LEN, DIM)


# speedup vs baseline: 6.9859x; 1.0100x over previous
import jax
import jax.numpy as jnp
import numpy as np
from jax import lax
from jax.experimental import pallas as pl
from jax.experimental.pallas import tpu as pltpu
from jax.experimental.pallas import tpu_sc as plsc

VOCAB = 100000
MAX_LEN = 200
DIM = 128
BATCH = 1024

NC = 2
NS = 16
NW = NC * NS
STAGE = 128
STAGES = (BATCH // NW) * MAX_LEN // STAGE
NBUF = 5
LOOKAHEAD = 3
HALF = STAGE // 2


def _build_pe():
    pos = np.arange(MAX_LEN, dtype=np.float32).reshape(-1, 1)
    div = np.power(
        10000.0, np.arange(0, DIM, 2, dtype=np.float32) / DIM)
    ang = (pos / div).astype(np.float32)
    pe = np.zeros((MAX_LEN, DIM), dtype=np.float32)
    pe[:, 0::2] = np.sin(ang)
    pe[:, 1::2] = np.cos(ang)
    return jnp.asarray(pe)


def _sc_body(x_hbm, pe_hbm, table_hbm, out_hbm,
             idx_all, r0, r1, r2, r3, r4, pe_v,
             g0, g1, g2, g3, g4, s0, s1, s2, s3, s4, psem):
    wid = lax.axis_index("s") * NC + lax.axis_index("c")
    rows = (r0, r1, r2, r3, r4)
    gsem = (g0, g1, g2, g3, g4)
    ssem = (s0, s1, s2, s3, s4)
    out_base = wid * (STAGES * STAGE)

    pe_cp = pltpu.async_copy(pe_hbm, pe_v, psem)
    pltpu.sync_copy(x_hbm.at[wid], idx_all)

    BLOCK = 25

    def block_body(k, carry):
        s0_dyn = k * (BLOCK * STAGE)
        pend_g = [None] * NBUF
        pend_s = [[] for _ in range(NBUF)]

        def fire_gather(j):
            b = j % NBUF
            pend_g[b] = pltpu.async_copy(
                table_hbm.at[idx_all.at[k * BLOCK + j]], rows[b], gsem[b])

        for j in range(LOOKAHEAD):
            fire_gather(j)
        for j in range(BLOCK):
            b = j % NBUF
            if j + LOOKAHEAD < BLOCK:
                b2 = (j + LOOKAHEAD) % NBUF
                for cp in pend_s[b2]:
                    cp.wait()
                pend_s[b2] = []
                fire_gather(j + LOOKAHEAD)
            pend_g[b].wait()

            rows_b = rows[b]
            base = (j * STAGE) % MAX_LEN
            cut = min(STAGE, MAX_LEN - base)

            def run_add(lo, hi):
                segs = []
                if lo < cut:
                    segs.append((lo, min(hi, cut), base))
                if hi > cut:
                    segs.append((max(lo, cut), hi, base - MAX_LEN))
                for (l, h, off) in segs:
                    @plsc.parallel_loop(l, h, unroll=2)
                    def add_body(i, off=off, rows_b=rows_b):
                        for cc in range(DIM // 16):
                            sl = pl.ds(cc * 16, 16)
                            plsc.addupdate(
                                rows_b.at[i, sl], pe_v[i + off, sl])

            for h in range(2):
                run_add(h * HALF, (h + 1) * HALF)
                pend_s[b].append(pltpu.async_copy(
                    rows_b.at[pl.ds(h * HALF, HALF)],
                    out_hbm.at[pl.ds(
                        out_base + s0_dyn + j * STAGE + h * HALF, HALF)],
                    ssem[b]))
        for b in range(NBUF):
            for cp in pend_s[b]:
                cp.wait()
        return carry

    pe_cp.wait()
    lax.fori_loop(0, STAGES // BLOCK, block_body, 0)


@jax.jit
def _run(x2, pe, table):
    mesh = plsc.VectorSubcoreMesh(
        core_axis_name="c", subcore_axis_name="s",
        num_cores=NC, num_subcores=NS)
    f = pl.kernel(
        _sc_body,
        out_type=jax.ShapeDtypeStruct((BATCH * MAX_LEN, DIM), jnp.float32),
        mesh=mesh,
        scratch_types=[
            pltpu.VMEM((STAGES, STAGE), jnp.int32),
            pltpu.VMEM((STAGE, DIM), jnp.float32),
            pltpu.VMEM((STAGE, DIM), jnp.float32),
            pltpu.VMEM((STAGE, DIM), jnp.float32),
            pltpu.VMEM((STAGE, DIM), jnp.float32),
            pltpu.VMEM((STAGE, DIM), jnp.float32),
            pltpu.VMEM((MAX_LEN, DIM), jnp.float32),
            pltpu.SemaphoreType.DMA,
            pltpu.SemaphoreType.DMA,
            pltpu.SemaphoreType.DMA,
            pltpu.SemaphoreType.DMA,
            pltpu.SemaphoreType.DMA,
            pltpu.SemaphoreType.DMA,
            pltpu.SemaphoreType.DMA,
            pltpu.SemaphoreType.DMA,
            pltpu.SemaphoreType.DMA,
            pltpu.SemaphoreType.DMA,
            pltpu.SemaphoreType.DMA,
        ],
    )
    return f(x2, pe, table)


def kernel(x, table):
    x2 = x.reshape(NW, STAGES, STAGE)
    pe = _build_pe()
    out = _run(x2, pe, table)
    return out.reshape(BATCH, MAX_LEN, DIM)


# speedup vs baseline: 7.1218x; 1.0195x over previous
import jax
import jax.numpy as jnp
import numpy as np
from jax import lax
from jax.experimental import pallas as pl
from jax.experimental.pallas import tpu as pltpu
from jax.experimental.pallas import tpu_sc as plsc

VOCAB = 100000
MAX_LEN = 200
DIM = 128
BATCH = 1024

NC = 2
NS = 16
NW = NC * NS
STAGE = 128
STAGES = (BATCH // NW) * MAX_LEN // STAGE
NBUF = 5
LOOKAHEAD = 3
HALF = STAGE // 2


def _build_pe():
    pos = np.arange(MAX_LEN, dtype=np.float32).reshape(-1, 1)
    div = np.power(
        10000.0, np.arange(0, DIM, 2, dtype=np.float32) / DIM)
    ang = (pos / div).astype(np.float32)
    pe = np.zeros((MAX_LEN, DIM), dtype=np.float32)
    pe[:, 0::2] = np.sin(ang)
    pe[:, 1::2] = np.cos(ang)
    return jnp.asarray(pe)


def _sc_body(x_hbm, pe_hbm, table_hbm, out_hbm,
             idx_all, r0, r1, r2, r3, r4, pe_v,
             g0, g1, g2, g3, g4, s0, s1, s2, s3, s4, psem):
    wid = lax.axis_index("s") * NC + lax.axis_index("c")
    rows = (r0, r1, r2, r3, r4)
    gsem = (g0, g1, g2, g3, g4)
    ssem = (s0, s1, s2, s3, s4)
    out_base = wid * (STAGES * STAGE)

    pe_cp = pltpu.async_copy(pe_hbm, pe_v, psem)
    pltpu.sync_copy(x_hbm.at[wid], idx_all)

    BLOCK = 25

    def block_body(k, carry):
        s0_dyn = k * (BLOCK * STAGE)
        pend_g = [None] * NBUF
        pend_s = [[] for _ in range(NBUF)]

        def fire_gather(j):
            b = j % NBUF
            pend_g[b] = pltpu.async_copy(
                table_hbm.at[idx_all.at[k * BLOCK + j]], rows[b], gsem[b])

        for j in range(LOOKAHEAD):
            fire_gather(j)
        for j in range(BLOCK):
            b = j % NBUF
            if j + LOOKAHEAD < BLOCK:
                b2 = (j + LOOKAHEAD) % NBUF
                for cp in pend_s[b2]:
                    cp.wait()
                pend_s[b2] = []
                fire_gather(j + LOOKAHEAD)
            pend_g[b].wait()

            rows_b = rows[b]
            base = (j * STAGE) % MAX_LEN
            cut = min(STAGE, MAX_LEN - base)

            def run_add(lo, hi):
                segs = []
                if lo < cut:
                    segs.append((lo, min(hi, cut), base))
                if hi > cut:
                    segs.append((max(lo, cut), hi, base - MAX_LEN))
                for (l, h, off) in segs:
                    @plsc.parallel_loop(l, h, unroll=2)
                    def add_body(i, off=off, rows_b=rows_b):
                        for cc in range(DIM // 16):
                            sl = pl.ds(cc * 16, 16)
                            plsc.addupdate(
                                rows_b.at[i, sl], pe_v[i + off, sl])

            run_add(0, STAGE)
            pend_s[b].append(pltpu.async_copy(
                rows_b,
                out_hbm.at[pl.ds(out_base + s0_dyn + j * STAGE, STAGE)],
                ssem[b]))
        for b in range(NBUF):
            for cp in pend_s[b]:
                cp.wait()
        return carry

    pe_cp.wait()
    lax.fori_loop(0, STAGES // BLOCK, block_body, 0)


@jax.jit
def _run(x2, pe, table):
    mesh = plsc.VectorSubcoreMesh(
        core_axis_name="c", subcore_axis_name="s",
        num_cores=NC, num_subcores=NS)
    f = pl.kernel(
        _sc_body,
        out_type=jax.ShapeDtypeStruct((BATCH * MAX_LEN, DIM), jnp.float32),
        mesh=mesh,
        scratch_types=[
            pltpu.VMEM((STAGES, STAGE), jnp.int32),
            pltpu.VMEM((STAGE, DIM), jnp.float32),
            pltpu.VMEM((STAGE, DIM), jnp.float32),
            pltpu.VMEM((STAGE, DIM), jnp.float32),
            pltpu.VMEM((STAGE, DIM), jnp.float32),
            pltpu.VMEM((STAGE, DIM), jnp.float32),
            pltpu.VMEM((MAX_LEN, DIM), jnp.float32),
            pltpu.SemaphoreType.DMA,
            pltpu.SemaphoreType.DMA,
            pltpu.SemaphoreType.DMA,
            pltpu.SemaphoreType.DMA,
            pltpu.SemaphoreType.DMA,
            pltpu.SemaphoreType.DMA,
            pltpu.SemaphoreType.DMA,
            pltpu.SemaphoreType.DMA,
            pltpu.SemaphoreType.DMA,
            pltpu.SemaphoreType.DMA,
            pltpu.SemaphoreType.DMA,
        ],
    )
    return f(x2, pe, table)


def kernel(x, table):
    x2 = x.reshape(NW, STAGES, STAGE)
    pe = _build_pe()
    out = _run(x2, pe, table)
    return out.reshape(BATCH, MAX_LEN, DIM)
